# baseline (speedup 1.0000x reference)
import functools

import jax
import jax.numpy as jnp
from jax import lax
from jax.experimental import pallas as pl
from jax.experimental.pallas import tpu as pltpu
from jax.experimental.pallas import tpu_sc as plsc

N = 10000
E = 160000
D_IN = 256
D_HID = 512
D_CLS = 128

C = 128
EP = 180224
NCHUNK = EP // C
NTILES = 16
RA = 10112
ZROWS = RA // NTILES

_mesh = plsc.VectorSubcoreMesh(core_axis_name="c", subcore_axis_name="s")


def _fill_const(ref, rows, width, value):
    @pl.loop(0, rows)
    def _(r):
        @pl.loop(0, width, step=16)
        def _(c):
            ref[r, pl.ds(c, 16)] = jnp.full((16,), value, jnp.float32)


def _zero_acc_rows(zsrc, acc, sid):
    zb = sid * ZROWS
    for off in range(0, ZROWS, 128):
        n = min(128, ZROWS - off)
        pltpu.sync_copy(zsrc.at[pl.ds(0, n)], acc.at[pl.ds(zb + off, n)])


def _unpack_idx(packed, k, srcl, dstl, b):
    @pl.loop(0, C, step=16)
    def _(c):
        v = packed[k, pl.ds(c, 16)]
        srcl[b, pl.ds(c, 16)] = v & 0xFFFF
        dstl[b, pl.ds(c, 16)] = v >> 16


def _run_edges(slab_hbm, packed, srcl, dstl, rows, acc, gsem, ssem, cpt, koff):
    _unpack_idx(packed, koff + 0, srcl, dstl, 0)

    def do_chunk(b, k, last):
        pltpu.async_copy(slab_hbm.at[srcl.at[b]], rows.at[0], gsem[0]).wait()
        sc = pltpu.async_copy(rows.at[0], acc.at[dstl.at[b]], ssem[0],
                              add=True)
        if not last:
            _unpack_idx(packed, koff + k + 1, srcl, dstl, 1 - b)
        sc.wait()

    @pl.loop(0, cpt // 2)
    def _(t):
        do_chunk(0, 2 * t, False)
        do_chunk(1, 2 * t + 1, False)


DW = 128
CPT_DEG = NCHUNK // (2 * NTILES)


@functools.partial(
    pl.kernel,
    out_type=jax.ShapeDtypeStruct((2, RA, DW), jnp.float32),
    mesh=_mesh,
    scratch_types=[
        pltpu.VMEM((2, C), jnp.int32),
        pltpu.VMEM((C, DW), jnp.float32),
        pltpu.VMEM_SHARED((RA, DW), jnp.float32),
        pltpu.SemaphoreType.DMA,
        pltpu.SemaphoreType.DMA,
        pltpu.SemaphoreType.DMA,
        pltpu.SemaphoreType.DMA,
    ],
)
def _deg_kernel(dstp_hbm, out_hbm, dstv, ones, acc, i0, i1, s0, s1):
    cid = lax.axis_index("c")
    sid = lax.axis_index("s")
    idst = [i0, i1]
    ssem = [s0, s1]
    basechunk = cid * (NCHUNK // 2) + sid * CPT_DEG
    _fill_const(ones, C, DW, 0.0)
    _zero_acc_rows(ones, acc, sid)
    _fill_const(ones, C, DW, 1.0)
    plsc.subcore_barrier()

    def eb(k):
        return (basechunk + k) * C

    def do_chunk(b, k, first):
        if not first:
            pltpu.make_async_copy(ones, acc.at[dstv.at[b]], ssem[b]).wait()
        pltpu.async_copy(dstp_hbm.at[pl.ds(eb(k), C)], dstv.at[b], idst[b])
        pltpu.make_async_copy(dstp_hbm.at[pl.ds(0, C)], dstv.at[b],
                              idst[b]).wait()
        pltpu.async_copy(ones, acc.at[dstv.at[b]], ssem[b], add=True)

    do_chunk(0, 0, True)
    do_chunk(1, 1, True)

    @pl.loop(1, CPT_DEG // 2)
    def _(t):
        do_chunk(0, 2 * t, False)
        do_chunk(1, 2 * t + 1, False)

    for b in range(2):
        pltpu.make_async_copy(ones, acc.at[dstv.at[b]], ssem[b]).wait()

    plsc.subcore_barrier()
    pltpu.sync_copy(acc.at[pl.ds(sid * ZROWS, ZROWS)],
                    out_hbm.at[cid].at[pl.ds(sid * ZROWS, ZROWS)])


CPT_L1 = NCHUNK // NTILES
CPT_L2 = NCHUNK // (2 * NTILES)


def _prop_scratch():
    return [
        pltpu.VMEM((CPT_L1, C), jnp.int32),
        pltpu.VMEM((2, C), jnp.int32),
        pltpu.VMEM((2, C), jnp.int32),
        pltpu.VMEM((2, C, 128), jnp.float32),
        pltpu.VMEM_SHARED((RA, 128), jnp.float32),
        pltpu.SemaphoreType.DMA,
        pltpu.SemaphoreType.DMA,
        pltpu.SemaphoreType.DMA,
        pltpu.SemaphoreType.DMA,
    ]


@functools.partial(
    pl.kernel,
    out_type=jax.ShapeDtypeStruct((4, RA, 128), jnp.float32),
    mesh=_mesh,
    scratch_types=_prop_scratch(),
)
def _prop_l1(xws_hbm, pidx_hbm, out_hbm, packed, srcl, dstl, rows, acc,
             g0, g1, s0, s1):
    cid = lax.axis_index("c")
    sid = lax.axis_index("s")
    gsem, ssem = [g0, g1], [s0, s1]
    pltpu.sync_copy(pidx_hbm.at[pl.ds(sid * CPT_L1, CPT_L1)], packed)

    for j in range(2):
        slice_id = cid * 2 + j
        _fill_const(rows.at[0], C, 128, 0.0)
        _zero_acc_rows(rows.at[0], acc, sid)
        plsc.subcore_barrier()
        _run_edges(xws_hbm.at[slice_id], packed, srcl, dstl, rows, acc,
                   gsem, ssem, CPT_L1, 0)
        plsc.subcore_barrier()
        wb = sid * ZROWS
        pltpu.sync_copy(acc.at[pl.ds(wb, ZROWS)],
                        out_hbm.at[slice_id].at[pl.ds(wb, ZROWS)])
        plsc.subcore_barrier()


@functools.partial(
    pl.kernel,
    out_type=jax.ShapeDtypeStruct((2, RA, D_CLS), jnp.float32),
    mesh=_mesh,
    scratch_types=_prop_scratch(),
)
def _prop_l2(xws_hbm, pidx_hbm, out_hbm, packed, srcl, dstl, rows, acc,
             g0, g1, s0, s1):
    cid = lax.axis_index("c")
    sid = lax.axis_index("s")
    gsem, ssem = [g0, g1], [s0, s1]
    pltpu.sync_copy(pidx_hbm.at[pl.ds(sid * CPT_L1, CPT_L1)], packed)
    _fill_const(rows.at[0], C, 128, 0.0)
    _zero_acc_rows(rows.at[0], acc, sid)
    plsc.subcore_barrier()
    _run_edges(xws_hbm, packed, srcl, dstl, rows, acc, gsem, ssem, CPT_L2,
               cid * CPT_L2)
    plsc.subcore_barrier()
    wb = sid * ZROWS
    pltpu.sync_copy(acc.at[pl.ds(wb, ZROWS)],
                    out_hbm.at[cid].at[pl.ds(wb, ZROWS)])


def _dinv_of(deg_blk):
    d = deg_blk[0, :, 0:1] + deg_blk[1, :, 0:1]
    return lax.rsqrt(jnp.maximum(d, 1e-12))


def _mm1_body(x_ref, w_ref, deg_ref, out_ref):
    acc = jnp.dot(x_ref[...], w_ref[...], preferred_element_type=jnp.float32)
    acc = acc * _dinv_of(deg_ref[...])
    for s in range(4):
        out_ref[s] = acc[:, s * 128:(s + 1) * 128]


def _mm2_body(h1_ref, w2_ref, deg_ref, out_ref):
    dinv = _dinv_of(deg_ref[...])
    acc = jnp.zeros((h1_ref.shape[1], 128), jnp.float32)
    for s in range(4):
        h = jnp.maximum(h1_ref[s] * dinv, 0.0)
        acc = acc + jnp.dot(h, w2_ref[s], preferred_element_type=jnp.float32)
    acc = acc * dinv
    out_ref[...] = acc


def _final_body(h2_ref, deg_ref, h3_ref, sm_ref):
    dinv = _dinv_of(deg_ref[...])
    h3 = (h2_ref[0] + h2_ref[1]) * dinv
    m = jnp.max(h3, axis=1, keepdims=True)
    e = jnp.exp(h3 - m)
    h3_ref[...] = h3
    sm_ref[...] = e / jnp.sum(e, axis=1, keepdims=True)


BN = 1000


def _mm1(x, W1, deg):
    return pl.pallas_call(
        _mm1_body,
        grid=(N // BN,),
        in_specs=[
            pl.BlockSpec((BN, D_IN), lambda i: (i, 0)),
            pl.BlockSpec((D_IN, D_HID), lambda i: (0, 0)),
            pl.BlockSpec((2, BN, DW), lambda i: (0, i, 0)),
        ],
        out_specs=pl.BlockSpec((4, BN, 128), lambda i: (0, i, 0)),
        out_shape=jax.ShapeDtypeStruct((4, N, 128), jnp.float32),
    )(x, W1, deg)


def _mm2(h1raw, W2r, deg):
    return pl.pallas_call(
        _mm2_body,
        grid=(N // BN,),
        in_specs=[
            pl.BlockSpec((4, BN, 128), lambda i: (0, i, 0)),
            pl.BlockSpec((4, 128, 128), lambda i: (0, 0, 0)),
            pl.BlockSpec((2, BN, DW), lambda i: (0, i, 0)),
        ],
        out_specs=pl.BlockSpec((BN, D_CLS), lambda i: (i, 0)),
        out_shape=jax.ShapeDtypeStruct((N, D_CLS), jnp.float32),
    )(h1raw, W2r, deg)


def _final(h2raw, deg):
    return pl.pallas_call(
        _final_body,
        grid=(N // BN,),
        in_specs=[
            pl.BlockSpec((2, BN, D_CLS), lambda i: (0, i, 0)),
            pl.BlockSpec((2, BN, DW), lambda i: (0, i, 0)),
        ],
        out_specs=(
            pl.BlockSpec((BN, D_CLS), lambda i: (i, 0)),
            pl.BlockSpec((BN, D_CLS), lambda i: (i, 0)),
        ),
        out_shape=(
            jax.ShapeDtypeStruct((N, D_CLS), jnp.float32),
            jax.ShapeDtypeStruct((N, D_CLS), jnp.float32),
        ),
    )(h2raw, deg)


def kernel(x, edge_index, batch_index, W1, W2):
    loop = jnp.arange(N, dtype=jnp.int32)
    npad = EP - E - N
    srcp = jnp.concatenate([edge_index[0], loop,
                            jnp.zeros((npad,), jnp.int32)])
    dstp = jnp.concatenate([edge_index[1], loop,
                            N + (jnp.arange(npad, dtype=jnp.int32) % (RA - N))])
    pidx = (srcp | (dstp << 16)).reshape(NCHUNK, C)

    deg = _deg_kernel(dstp)
    xw1s = _mm1(x, W1, deg)
    h1raw = _prop_l1(xw1s, pidx)
    xw2s = _mm2(h1raw, W2.reshape(4, 128, D_CLS), deg)
    h2raw = _prop_l2(xw2s, pidx)
    hidden3, output = _final(h2raw, deg)
    return (hidden3, output)


# speedup vs baseline: 1.7549x; 1.7549x over previous
import functools

import jax
import jax.numpy as jnp
from jax import lax
from jax.experimental import pallas as pl
from jax.experimental.pallas import tpu as pltpu
from jax.experimental.pallas import tpu_sc as plsc

N = 10000
E = 160000
D_IN = 256
D_HID = 512
D_CLS = 128

C = 128
EP = 172032
NCHUNK = EP // C
NTILES = 16
CHUNKS_PER_TILE = NCHUNK // NTILES
RA = 10240
ZROWS = RA // NTILES
WROWS = RA // NTILES

_mesh = plsc.VectorSubcoreMesh(core_axis_name="c", subcore_axis_name="s")


def _fill_const(ref, rows, width, value):
    @pl.loop(0, rows)
    def _(r):
        @pl.loop(0, width, step=16)
        def _(c):
            ref[r, pl.ds(c, 16)] = jnp.full((16,), value, jnp.float32)


DW = 128


@functools.partial(
    pl.kernel,
    out_type=jax.ShapeDtypeStruct((RA, DW), jnp.float32),
    mesh=_mesh,
    scratch_types=[
        pltpu.VMEM((2, C), jnp.int32),
        pltpu.VMEM((C, DW), jnp.float32),
        pltpu.VMEM((128, DW), jnp.float32),
        pltpu.VMEM_SHARED((RA, DW), jnp.float32),
        pltpu.SemaphoreType.DMA,
    ],
)
def _deg_kernel(dstp_hbm, out_hbm, dstv, ones, zbuf, acc, sem):
    cid = lax.axis_index("c")
    sid = lax.axis_index("s")

    @pl.when(cid == 0)
    def _():
        _fill_const(ones, C, DW, 1.0)
        _fill_const(zbuf, 128, DW, 0.0)
        zb = sid * ZROWS
        for off in range(0, ZROWS, 128):
            n = min(128, ZROWS - off)
            pltpu.sync_copy(zbuf.at[pl.ds(0, n)], acc.at[pl.ds(zb + off, n)])
        plsc.subcore_barrier()

        @pl.loop(0, CHUNKS_PER_TILE)
        def _(k):
            eb = (sid * CHUNKS_PER_TILE + k) * C
            pltpu.sync_copy(dstp_hbm.at[pl.ds(eb, C)], dstv.at[0])
            pltpu.sync_copy(ones, acc.at[dstv.at[0]], add=True)

        plsc.subcore_barrier()
        pltpu.sync_copy(acc.at[pl.ds(sid * ZROWS, ZROWS)],
                        out_hbm.at[pl.ds(sid * ZROWS, ZROWS)])


def _make_prop(S, W):
    S_per_core = S // 2

    @functools.partial(
        pl.kernel,
        out_type=jax.ShapeDtypeStruct((S, RA, W), jnp.float32),
        mesh=_mesh,
        scratch_types=[
            pltpu.VMEM((2, C), jnp.int32),
            pltpu.VMEM((2, C), jnp.int32),
            pltpu.VMEM((C, W), jnp.float32),
            pltpu.VMEM((128, W), jnp.float32),
            pltpu.VMEM_SHARED((RA, W), jnp.float32),
            pltpu.SemaphoreType.DMA,
        ],
    )
    def _prop(xws_hbm, srcp_hbm, dstp_hbm, out_hbm, srcv, dstv, rows, zbuf,
              acc, sem):
        cid = lax.axis_index("c")
        sid = lax.axis_index("s")
        _fill_const(zbuf, 128, W, 0.0)

        for j in range(S_per_core):
            slice_id = cid * S_per_core + j
            zb = sid * ZROWS
            for off in range(0, ZROWS, 128):
                n = min(128, ZROWS - off)
                pltpu.sync_copy(zbuf.at[pl.ds(0, n)], acc.at[pl.ds(zb + off, n)])
            plsc.subcore_barrier()

            @pl.loop(0, CHUNKS_PER_TILE)
            def _(k):
                eb = (sid * CHUNKS_PER_TILE + k) * C
                pltpu.sync_copy(srcp_hbm.at[pl.ds(eb, C)], srcv.at[0])
                pltpu.sync_copy(dstp_hbm.at[pl.ds(eb, C)], dstv.at[0])
                pltpu.async_copy(xws_hbm.at[slice_id].at[srcv.at[0]], rows,
                                 sem).wait()
                pltpu.sync_copy(rows, acc.at[dstv.at[0]], add=True)

            plsc.subcore_barrier()
            wb = sid * WROWS
            pltpu.sync_copy(acc.at[pl.ds(wb, WROWS)],
                            out_hbm.at[slice_id].at[pl.ds(wb, WROWS)])
            plsc.subcore_barrier()

    return _prop


_prop_l1 = _make_prop(4, 128)


@functools.partial(
    pl.kernel,
    out_type=jax.ShapeDtypeStruct((2, RA, D_CLS), jnp.float32),
    mesh=_mesh,
    scratch_types=[
        pltpu.VMEM((2, C), jnp.int32),
        pltpu.VMEM((2, C), jnp.int32),
        pltpu.VMEM((C, D_CLS), jnp.float32),
        pltpu.VMEM((128, D_CLS), jnp.float32),
        pltpu.VMEM_SHARED((RA, D_CLS), jnp.float32),
        pltpu.SemaphoreType.DMA,
    ],
)
def _prop_l2(xws_hbm, srcp_hbm, dstp_hbm, out_hbm, srcv, dstv, rows, zbuf,
             acc, sem):
    cid = lax.axis_index("c")
    sid = lax.axis_index("s")
    _fill_const(zbuf, 128, D_CLS, 0.0)
    zb = sid * ZROWS
    for off in range(0, ZROWS, 128):
        n = min(128, ZROWS - off)
        pltpu.sync_copy(zbuf.at[pl.ds(0, n)], acc.at[pl.ds(zb + off, n)])
    plsc.subcore_barrier()

    half = NCHUNK // 2

    @pl.loop(0, half // NTILES)
    def _(k):
        eb = (cid * half + sid * (half // NTILES) + k) * C
        pltpu.sync_copy(srcp_hbm.at[pl.ds(eb, C)], srcv.at[0])
        pltpu.sync_copy(dstp_hbm.at[pl.ds(eb, C)], dstv.at[0])
        pltpu.async_copy(xws_hbm.at[srcv.at[0]], rows, sem).wait()
        pltpu.sync_copy(rows, acc.at[dstv.at[0]], add=True)

    plsc.subcore_barrier()
    wb = sid * WROWS
    pltpu.sync_copy(acc.at[pl.ds(wb, WROWS)],
                    out_hbm.at[cid].at[pl.ds(wb, WROWS)])


def _dinv_of(deg_blk):
    return lax.rsqrt(jnp.maximum(deg_blk[:, 0:1], 1e-12))


def _mm1_body(x_ref, w_ref, deg_ref, out_ref):
    acc = jnp.dot(x_ref[...], w_ref[...], preferred_element_type=jnp.float32)
    acc = acc * _dinv_of(deg_ref[...])
    for s in range(4):
        out_ref[s] = acc[:, s * 128:(s + 1) * 128]


def _mm2_body(h1_ref, w2_ref, deg_ref, out_ref):
    dinv = _dinv_of(deg_ref[...])
    acc = jnp.zeros((h1_ref.shape[1], 128), jnp.float32)
    for s in range(4):
        h = jnp.maximum(h1_ref[s] * dinv, 0.0)
        acc = acc + jnp.dot(h, w2_ref[s], preferred_element_type=jnp.float32)
    acc = acc * dinv
    out_ref[...] = acc


def _final_body(h2_ref, deg_ref, h3_ref, sm_ref):
    dinv = _dinv_of(deg_ref[...])
    h3 = (h2_ref[0] + h2_ref[1]) * dinv
    m = jnp.max(h3, axis=1, keepdims=True)
    e = jnp.exp(h3 - m)
    h3_ref[...] = h3
    sm_ref[...] = e / jnp.sum(e, axis=1, keepdims=True)


BN = 1000


def _mm1(x, W1, deg):
    return pl.pallas_call(
        _mm1_body,
        grid=(N // BN,),
        in_specs=[
            pl.BlockSpec((BN, D_IN), lambda i: (i, 0)),
            pl.BlockSpec((D_IN, D_HID), lambda i: (0, 0)),
            pl.BlockSpec((BN, DW), lambda i: (i, 0)),
        ],
        out_specs=pl.BlockSpec((4, BN, 128), lambda i: (0, i, 0)),
        out_shape=jax.ShapeDtypeStruct((4, N, 128), jnp.float32),
    )(x, W1, deg)


def _mm2(h1raw, W2r, deg):
    return pl.pallas_call(
        _mm2_body,
        grid=(N // BN,),
        in_specs=[
            pl.BlockSpec((4, BN, 128), lambda i: (0, i, 0)),
            pl.BlockSpec((4, 128, 128), lambda i: (0, 0, 0)),
            pl.BlockSpec((BN, DW), lambda i: (i, 0)),
        ],
        out_specs=pl.BlockSpec((BN, D_CLS), lambda i: (i, 0)),
        out_shape=jax.ShapeDtypeStruct((N, D_CLS), jnp.float32),
    )(h1raw, W2r, deg)


def _final(h2raw, deg):
    return pl.pallas_call(
        _final_body,
        grid=(N // BN,),
        in_specs=[
            pl.BlockSpec((2, BN, D_CLS), lambda i: (0, i, 0)),
            pl.BlockSpec((BN, DW), lambda i: (i, 0)),
        ],
        out_specs=(
            pl.BlockSpec((BN, D_CLS), lambda i: (i, 0)),
            pl.BlockSpec((BN, D_CLS), lambda i: (i, 0)),
        ),
        out_shape=(
            jax.ShapeDtypeStruct((N, D_CLS), jnp.float32),
            jax.ShapeDtypeStruct((N, D_CLS), jnp.float32),
        ),
    )(h2raw, deg)


def kernel(x, edge_index, batch_index, W1, W2):
    loop = jnp.arange(N, dtype=jnp.int32)
    npad = EP - E - N
    srcp = jnp.concatenate([edge_index[0], loop,
                            jnp.zeros((npad,), jnp.int32)])
    dstp = jnp.concatenate([edge_index[1], loop,
                            jnp.full((npad,), N, jnp.int32)])

    deg = _deg_kernel(dstp)
    xw1s = _mm1(x, W1, deg)
    h1raw = _prop_l1(xw1s, srcp, dstp)
    xw2s = _mm2(h1raw, W2.reshape(4, 128, D_CLS), deg)
    h2raw = _prop_l2(xw2s, srcp, dstp)
    hidden3, output = _final(h2raw, deg)
    return (hidden3, output)
